# baseline (speedup 1.0000x reference)
import functools

import jax
import jax.numpy as jnp
from jax import lax
from jax.experimental import pallas as pl
from jax.experimental.pallas import tpu as pltpu
from jax.experimental.pallas import tpu_sc as plsc

ENTITY_SIZE = 1000000
EMB = 64
BATCH = 16384
TOTAL = 2 * BATCH
NENT = 2 * TOTAL
NREL = TOTAL

NC = 2
NS = 16
NW = NC * NS
L = 16

ENT_PER_W = NENT // NW
REL_PER_W = NREL // NW

CHUNK = 1024
CSH = 10
NSL = CHUNK // 128
TAIL_K = ENTITY_SIZE // CHUNK
TAIL_SL = 5
TAIL_BASE = ENTITY_SIZE - TAIL_SL * 128
BATCH_G = 8
BATCH_SH = 3
BROWS = BATCH_G * L
NROW = 128
SCALE_COL = EMB
DUMP = NENT + NREL
STAGE_ROWS = DUMP + 16


def _rsqrt_nr(x):
    i = plsc.bitcast(x, jnp.int32)
    i = jnp.int32(0x5F3759DF) - lax.shift_right_logical(i, 1)
    y = plsc.bitcast(i, jnp.float32)
    half = x * jnp.float32(0.5)
    for _ in range(3):
        y = y * (jnp.float32(1.5) - half * y * y)
    return y


def _extract(tab_hbm, tail_hbm, ids_v, slots_v, stage_hbm, chunk_v, row_v,
             slot_v, sem, csem, n_lookups, is_entity, lane):
    zeros = jnp.zeros((L,), jnp.float32)
    ones = jnp.full((L,), 1.0, jnp.float32)
    dump_vec = jnp.full((L,), DUMP, jnp.int32)

    for b in range(2):
        for g in range(BATCH_G):
            slot_v[b, pl.ds(g * L, L)] = dump_vec
    for b in range(2):
        pltpu.async_copy(row_v.at[b], stage_hbm.at[slot_v.at[b]], sem)

    def cond(carry):
        pos, _, _ = carry
        return pos < n_lookups

    def body(carry):
        pos, cur_k, it = carry
        gc = it & (BATCH_G - 1)
        p = (it >> BATCH_SH) & 1
        ev16 = ids_v[pl.ds(pos, L)]
        sv16 = slots_v[pl.ds(pos, L)]
        e0 = ev16[0]
        k = e0 >> CSH
        tail = k >= TAIL_K
        base_main = k * CHUNK
        base = jnp.where(tail, TAIL_BASE, base_main)
        win = jnp.where(tail, TAIL_SL * 128, CHUNK)

        @pl.when(k != cur_k)
        def _():
            @pl.when(tail)
            def _():
                cps = [
                    pltpu.async_copy(
                        tail_hbm.at[:, pl.ds(c * 128, 128)],
                        chunk_v.at[c], csem,
                    )
                    for c in range(TAIL_SL)
                ]
                for cp in cps:
                    cp.wait()

            @pl.when(jnp.logical_not(tail))
            def _():
                cps = []
                for c in range(NSL):
                    start = pl.multiple_of(base_main + c * 128, 128)
                    cps.append(pltpu.async_copy(
                        tab_hbm.at[:, pl.ds(start, 128)], chunk_v.at[c], csem
                    ))
                for cp in cps:
                    cp.wait()

        lanes = pos + lane
        inb = lanes < n_lookups
        e_v = ev16
        valid = inb & (e_v < base + win)
        slot = jnp.where(valid, sv16, dump_vec)
        o = jnp.where(valid, e_v - base, 0)
        oc = o >> 7
        ol = o & 127

        @pl.when(gc == 0)
        def _():
            pltpu.make_async_copy(
                row_v.at[0], stage_hbm.at[slot_v.at[0]], sem
            ).wait()
            for g in range(BATCH_G):
                slot_v[p, pl.ds(g * L, L)] = dump_vec

        rb = row_v.at[p]
        rlane = gc * L + lane

        def dims(db, acc):
            for u in range(4):
                d = db * 4 + u
                dv = jnp.full((L,), 0, jnp.int32) + d
                col = plsc.load_gather(chunk_v, [oc, dv, ol])
                plsc.store_scatter(rb, [rlane, dv], col)
                acc = acc + col * col
            return acc

        acc = lax.fori_loop(0, EMB // 4, dims, zeros)
        if is_entity:
            scale = jnp.where(
                e_v == ENTITY_SIZE - 1, ones, _rsqrt_nr(acc)
            )
        else:
            scale = ones
        plsc.store_scatter(
            rb, [rlane, jnp.full((L,), SCALE_COL, jnp.int32)], scale
        )
        plsc.store_scatter(slot_v.at[p], [rlane], slot)

        @pl.when(gc == BATCH_G - 1)
        def _():
            pltpu.async_copy(rb, stage_hbm.at[slot_v.at[p]], sem)

        nvalid = jnp.sum(valid.astype(jnp.int32))
        return pos + nvalid, k, it + 1

    pos, cur_k, it = lax.while_loop(
        cond, body, (jnp.int32(0), jnp.int32(-1), jnp.int32(0))
    )

    pf = (it >> BATCH_SH) & 1
    pltpu.async_copy(row_v.at[pf], stage_hbm.at[slot_v.at[pf]], sem)
    for b in range(2):
        pltpu.make_async_copy(
            row_v.at[0], stage_hbm.at[slot_v.at[0]], sem
        ).wait()

    @pl.when((it & (BATCH_G - 1)) == 0)
    def _():
        pltpu.make_async_copy(
            row_v.at[0], stage_hbm.at[slot_v.at[0]], sem
        ).wait()


def _phase1_body(eids_hbm, eslots_hbm, rids_hbm, rslots_hbm,
                 ent_hbm, rel_hbm, ent_tail_hbm, rel_tail_hbm, stage_hbm,
                 eid_v, eslot_v, rid_v, rslot_v, chunk_v, row_v, slot_v,
                 sem, csem):
    wid = lax.axis_index("s") * NC + lax.axis_index("c")
    lane = lax.broadcasted_iota(jnp.int32, (L,), 0)

    cps = [
        pltpu.async_copy(eids_hbm.at[pl.ds(wid * ENT_PER_W, ENT_PER_W)],
                         eid_v.at[pl.ds(0, ENT_PER_W)], csem),
        pltpu.async_copy(eslots_hbm.at[pl.ds(wid * ENT_PER_W, ENT_PER_W)],
                         eslot_v.at[pl.ds(0, ENT_PER_W)], csem),
        pltpu.async_copy(rids_hbm.at[pl.ds(wid * REL_PER_W, REL_PER_W)],
                         rid_v.at[pl.ds(0, REL_PER_W)], csem),
        pltpu.async_copy(rslots_hbm.at[pl.ds(wid * REL_PER_W, REL_PER_W)],
                         rslot_v.at[pl.ds(0, REL_PER_W)], csem),
    ]
    for cp in cps:
        cp.wait()

    _extract(ent_hbm, ent_tail_hbm, eid_v, eslot_v, stage_hbm, chunk_v,
             row_v, slot_v, sem, csem, ENT_PER_W, True, lane)
    _extract(rel_hbm, rel_tail_hbm, rid_v, rslot_v, stage_hbm, chunk_v,
             row_v, slot_v, sem, csem, REL_PER_W, False, lane)


TRIP_PER_W = TOTAL // NW
P2CH = 128


def _phase2_body(stage_hbm, out_hbm, h_v, r_v, t_v, out_v, p2sem):
    wid = lax.axis_index("s") * NC + lax.axis_index("c")
    lane = lax.broadcasted_iota(jnp.int32, (L,), 0)
    zeros = jnp.zeros((L,), jnp.float32)

    for c in range(TRIP_PER_W // P2CH):
        tbase = wid * TRIP_PER_W + c * P2CH
        cps = [
            pltpu.async_copy(stage_hbm.at[pl.ds(tbase, P2CH)], h_v, p2sem),
            pltpu.async_copy(stage_hbm.at[pl.ds(TOTAL + tbase, P2CH)], t_v,
                             p2sem),
            pltpu.async_copy(stage_hbm.at[pl.ds(2 * TOTAL + tbase, P2CH)],
                             r_v, p2sem),
        ]
        for cp in cps:
            cp.wait()

        def group(g, carry):
            rows = g * L + lane
            scol = jnp.full((L,), SCALE_COL, jnp.int32)
            sh = plsc.load_gather(h_v, [rows, scol])
            st = plsc.load_gather(t_v, [rows, scol])

            def dims(db, acc):
                for u in range(4):
                    dv = jnp.full((L,), 0, jnp.int32) + (db * 4 + u)
                    gh = plsc.load_gather(h_v, [rows, dv])
                    gr = plsc.load_gather(r_v, [rows, dv])
                    gt = plsc.load_gather(t_v, [rows, dv])
                    d = gh * sh + gr - gt * st
                    acc = acc + d * d
                return acc

            acc = lax.fori_loop(0, EMB // 4, dims, zeros)
            dist = jnp.where(acc > 0, acc * _rsqrt_nr(acc), zeros)
            out_v[pl.ds(c * P2CH + g * L, L)] = dist
            return carry

        lax.fori_loop(0, P2CH // L, group, 0)

    pltpu.sync_copy(out_v, out_hbm.at[pl.ds(wid * TRIP_PER_W, TRIP_PER_W)])


@jax.jit
def _trans_e(eids, eslots, rids, rslots, ent_t, rel_t, ent_tail, rel_tail):
    mesh = plsc.VectorSubcoreMesh(core_axis_name="c", subcore_axis_name="s")
    params = pltpu.CompilerParams(
        needs_layout_passes=False, use_tc_tiling_on_sc=True
    )
    stage = functools.partial(
        pl.kernel,
        out_type=jax.ShapeDtypeStruct((STAGE_ROWS, NROW), jnp.float32),
        mesh=mesh,
        scratch_types=[
            pltpu.VMEM((ENT_PER_W + L,), jnp.int32),
            pltpu.VMEM((ENT_PER_W + L,), jnp.int32),
            pltpu.VMEM((REL_PER_W + L,), jnp.int32),
            pltpu.VMEM((REL_PER_W + L,), jnp.int32),
            pltpu.VMEM((NSL, EMB, 128), jnp.float32),
            pltpu.VMEM((2, BROWS, NROW), jnp.float32),
            pltpu.VMEM((2, BROWS), jnp.int32),
            pltpu.SemaphoreType.DMA,
            pltpu.SemaphoreType.DMA,
        ],
        compiler_params=params,
    )(_phase1_body)(eids, eslots, rids, rslots, ent_t, rel_t,
                    ent_tail, rel_tail)

    return functools.partial(
        pl.kernel,
        out_type=jax.ShapeDtypeStruct((TOTAL,), jnp.float32),
        mesh=mesh,
        scratch_types=[
            pltpu.VMEM((P2CH, NROW), jnp.float32),
            pltpu.VMEM((P2CH, NROW), jnp.float32),
            pltpu.VMEM((P2CH, NROW), jnp.float32),
            pltpu.VMEM((TRIP_PER_W,), jnp.float32),
            pltpu.SemaphoreType.DMA,
        ],
        compiler_params=params,
    )(_phase2_body)(stage)


def kernel(positive_triplets, negative_triplets, entity_emb, relation_emb):
    trip = jnp.concatenate([positive_triplets, negative_triplets], axis=0)
    trip = trip.astype(jnp.int32)
    eids = jnp.concatenate([trip[:, 0], trip[:, 2]])
    rids = trip[:, 1]
    eorder = jnp.argsort(eids).astype(jnp.int32)
    rorder = jnp.argsort(rids).astype(jnp.int32)
    out = _trans_e(
        eids[eorder], eorder,
        rids[rorder], rorder + jnp.int32(2 * TOTAL),
        entity_emb.T, relation_emb.T,
        entity_emb[TAIL_BASE:].T, relation_emb[TAIL_BASE:].T,
    )
    return out[:BATCH], out[BATCH:]


# speedup vs baseline: 3.8103x; 3.8103x over previous
import functools

import jax
import jax.numpy as jnp
from jax import lax
from jax.experimental import pallas as pl
from jax.experimental.pallas import tpu as pltpu
from jax.experimental.pallas import tpu_sc as plsc

ENTITY_SIZE = 1000000
EMB = 64
BATCH = 16384
TOTAL = 2 * BATCH
NENT = 2 * TOTAL
NREL = TOTAL

NC = 2
NS = 16
NW = NC * NS
L = 16

ENT_PER_W = NENT // NW
REL_PER_W = NREL // NW

CHUNK = 1024
CSH = 10
NSL = CHUNK // 128
TAIL_K = ENTITY_SIZE // CHUNK
TAIL_SL = 5
TAIL_BASE = ENTITY_SIZE - TAIL_SL * 128

FLUSH = 128
NROW = 128
SCALE_COL = EMB
PRIME_BASE = NENT + NREL
STAGE_ROWS = PRIME_BASE + FLUSH


def _rsqrt_nr(x):
    i = plsc.bitcast(x, jnp.int32)
    i = jnp.int32(0x5F3759DF) - lax.shift_right_logical(i, 1)
    y = plsc.bitcast(i, jnp.float32)
    half = x * jnp.float32(0.5)
    for _ in range(3):
        y = y * (jnp.float32(1.5) - half * y * y)
    return y


def _extract(tab_hbm, tail_hbm, ids_v, stage_hbm, chunk_v, ring_v,
             sem, csem, n_lookups, obase, is_entity, lane):
    zeros = jnp.zeros((L,), jnp.float32)
    ones = jnp.full((L,), 1.0, jnp.float32)

    pltpu.async_copy(
        ring_v.at[pl.ds(0, FLUSH)],
        stage_hbm.at[pl.ds(PRIME_BASE, FLUSH)], sem,
    )

    def cond(carry):
        pos, _, _ = carry
        return pos < n_lookups

    def body(carry):
        pos, cur_k, nb = carry
        ev16 = ids_v[pl.ds(pos, L)]
        e0 = ev16[0]
        k = e0 >> CSH
        tail = k >= TAIL_K
        base_main = k * CHUNK
        base = jnp.where(tail, TAIL_BASE, base_main)
        win = jnp.where(tail, TAIL_SL * 128, CHUNK)

        @pl.when(k != cur_k)
        def _():
            @pl.when(tail)
            def _():
                cps = [
                    pltpu.async_copy(
                        tail_hbm.at[:, pl.ds(c * 128, 128)],
                        chunk_v.at[c], csem,
                    )
                    for c in range(TAIL_SL)
                ]
                for cp in cps:
                    cp.wait()

            @pl.when(jnp.logical_not(tail))
            def _():
                cps = []
                for c in range(NSL):
                    start = pl.multiple_of(base_main + c * 128, 128)
                    cps.append(pltpu.async_copy(
                        tab_hbm.at[:, pl.ds(start, 128)], chunk_v.at[c], csem
                    ))
                for cp in cps:
                    cp.wait()

        lanes = pos + lane
        inb = lanes < n_lookups
        e_v = ev16
        valid = inb & (e_v < base + win)
        o = jnp.where(valid, e_v - base, 0)
        oc = o >> 7
        ol = o & 127
        rrow = lanes & (2 * FLUSH - 1)

        def dims(db, acc):
            for u in range(4):
                d = db * 4 + u
                dv = jnp.full((L,), 0, jnp.int32) + d
                col = plsc.load_gather(chunk_v, [oc, dv, ol])
                plsc.store_scatter(ring_v, [rrow, dv], col, mask=valid)
                acc = acc + col * col
            return acc

        acc = lax.fori_loop(0, EMB // 4, dims, zeros)
        if is_entity:
            scale = jnp.where(e_v == ENTITY_SIZE - 1, ones, _rsqrt_nr(acc))
        else:
            scale = ones
        plsc.store_scatter(
            ring_v, [rrow, jnp.full((L,), SCALE_COL, jnp.int32)], scale,
            mask=valid,
        )

        nvalid = jnp.sum(valid.astype(jnp.int32))
        pos_new = pos + nvalid

        @pl.when(pos_new >= (nb + 1) * FLUSH)
        def _():
            pltpu.make_async_copy(
                ring_v.at[pl.ds(0, FLUSH)],
                stage_hbm.at[pl.ds(PRIME_BASE, FLUSH)], sem,
            ).wait()
            half = (nb & 1) * FLUSH
            pltpu.async_copy(
                ring_v.at[pl.ds(half, FLUSH)],
                stage_hbm.at[pl.ds(obase + nb * FLUSH, FLUSH)], sem,
            )

        nb_new = jnp.where(pos_new >= (nb + 1) * FLUSH, nb + 1, nb)
        return pos_new, k, nb_new

    lax.while_loop(cond, body, (jnp.int32(0), jnp.int32(-1), jnp.int32(0)))

    pltpu.make_async_copy(
        ring_v.at[pl.ds(0, FLUSH)],
        stage_hbm.at[pl.ds(PRIME_BASE, FLUSH)], sem,
    ).wait()


def _phase1_body(eids_hbm, rids_hbm, ent_hbm, rel_hbm,
                 ent_tail_hbm, rel_tail_hbm, stage_hbm,
                 eid_v, rid_v, chunk_v, ring_v, sem, csem):
    wid = lax.axis_index("s") * NC + lax.axis_index("c")
    lane = lax.broadcasted_iota(jnp.int32, (L,), 0)

    cps = [
        pltpu.async_copy(eids_hbm.at[pl.ds(wid * ENT_PER_W, ENT_PER_W)],
                         eid_v.at[pl.ds(0, ENT_PER_W)], csem),
        pltpu.async_copy(rids_hbm.at[pl.ds(wid * REL_PER_W, REL_PER_W)],
                         rid_v.at[pl.ds(0, REL_PER_W)], csem),
    ]
    for cp in cps:
        cp.wait()

    _extract(ent_hbm, ent_tail_hbm, eid_v, stage_hbm, chunk_v, ring_v,
             sem, csem, ENT_PER_W, wid * ENT_PER_W, True, lane)
    _extract(rel_hbm, rel_tail_hbm, rid_v, stage_hbm, chunk_v, ring_v,
             sem, csem, REL_PER_W, NENT + wid * REL_PER_W, False, lane)


TRIP_PER_W = TOTAL // NW
P2CH = 128


def _phase2_body(stage_hbm, hpos_hbm, tpos_hbm, rpos_hbm, out_hbm,
                 h_v, r_v, t_v, hp_v, tp_v, rp_v, out_v, p2sem):
    wid = lax.axis_index("s") * NC + lax.axis_index("c")
    lane = lax.broadcasted_iota(jnp.int32, (L,), 0)
    zeros = jnp.zeros((L,), jnp.float32)
    tb = wid * TRIP_PER_W

    cps = [
        pltpu.async_copy(hpos_hbm.at[pl.ds(tb, TRIP_PER_W)], hp_v, p2sem),
        pltpu.async_copy(tpos_hbm.at[pl.ds(tb, TRIP_PER_W)], tp_v, p2sem),
        pltpu.async_copy(rpos_hbm.at[pl.ds(tb, TRIP_PER_W)], rp_v, p2sem),
    ]
    for cp in cps:
        cp.wait()

    for c in range(TRIP_PER_W // P2CH):
        cps = [
            pltpu.async_copy(
                stage_hbm.at[hp_v.at[pl.ds(c * P2CH, P2CH)]], h_v, p2sem),
            pltpu.async_copy(
                stage_hbm.at[tp_v.at[pl.ds(c * P2CH, P2CH)]], t_v, p2sem),
            pltpu.async_copy(
                stage_hbm.at[rp_v.at[pl.ds(c * P2CH, P2CH)]], r_v, p2sem),
        ]
        for cp in cps:
            cp.wait()

        def group(g, carry):
            rows = g * L + lane
            scol = jnp.full((L,), SCALE_COL, jnp.int32)
            sh = plsc.load_gather(h_v, [rows, scol])
            st = plsc.load_gather(t_v, [rows, scol])

            def dims(db, acc):
                for u in range(4):
                    dv = jnp.full((L,), 0, jnp.int32) + (db * 4 + u)
                    gh = plsc.load_gather(h_v, [rows, dv])
                    gr = plsc.load_gather(r_v, [rows, dv])
                    gt = plsc.load_gather(t_v, [rows, dv])
                    d = gh * sh + gr - gt * st
                    acc = acc + d * d
                return acc

            acc = lax.fori_loop(0, EMB // 4, dims, zeros)
            dist = jnp.where(acc > 0, acc * _rsqrt_nr(acc), zeros)
            out_v[pl.ds(c * P2CH + g * L, L)] = dist
            return carry

        lax.fori_loop(0, P2CH // L, group, 0)

    pltpu.sync_copy(out_v, out_hbm.at[pl.ds(tb, TRIP_PER_W)])


@jax.jit
def _trans_e(eids_sorted, rids_sorted, hpos, tpos, rpos,
             ent_t, rel_t, ent_tail, rel_tail):
    mesh = plsc.VectorSubcoreMesh(core_axis_name="c", subcore_axis_name="s")
    params = pltpu.CompilerParams(
        needs_layout_passes=False, use_tc_tiling_on_sc=True
    )
    stage = functools.partial(
        pl.kernel,
        out_type=jax.ShapeDtypeStruct((STAGE_ROWS, NROW), jnp.float32),
        mesh=mesh,
        scratch_types=[
            pltpu.VMEM((ENT_PER_W + L,), jnp.int32),
            pltpu.VMEM((REL_PER_W + L,), jnp.int32),
            pltpu.VMEM((NSL, EMB, 128), jnp.float32),
            pltpu.VMEM((2 * FLUSH, NROW), jnp.float32),
            pltpu.SemaphoreType.DMA,
            pltpu.SemaphoreType.DMA,
        ],
        compiler_params=params,
    )(_phase1_body)(eids_sorted, rids_sorted, ent_t, rel_t,
                    ent_tail, rel_tail)

    return functools.partial(
        pl.kernel,
        out_type=jax.ShapeDtypeStruct((TOTAL,), jnp.float32),
        mesh=mesh,
        scratch_types=[
            pltpu.VMEM((P2CH, NROW), jnp.float32),
            pltpu.VMEM((P2CH, NROW), jnp.float32),
            pltpu.VMEM((P2CH, NROW), jnp.float32),
            pltpu.VMEM((TRIP_PER_W,), jnp.int32),
            pltpu.VMEM((TRIP_PER_W,), jnp.int32),
            pltpu.VMEM((TRIP_PER_W,), jnp.int32),
            pltpu.VMEM((TRIP_PER_W,), jnp.float32),
            pltpu.SemaphoreType.DMA,
        ],
        compiler_params=params,
    )(_phase2_body)(stage, hpos, tpos, rpos)


def kernel(positive_triplets, negative_triplets, entity_emb, relation_emb):
    trip = jnp.concatenate([positive_triplets, negative_triplets], axis=0)
    trip = trip.astype(jnp.int32)
    eids = jnp.concatenate([trip[:, 0], trip[:, 2]])
    rids = trip[:, 1]
    eorder = jnp.argsort(eids).astype(jnp.int32)
    rorder = jnp.argsort(rids).astype(jnp.int32)
    arange_e = jnp.arange(NENT, dtype=jnp.int32)
    arange_r = jnp.arange(NREL, dtype=jnp.int32)
    inv_e = jnp.zeros((NENT,), jnp.int32).at[eorder].set(arange_e)
    inv_r = jnp.zeros((NREL,), jnp.int32).at[rorder].set(arange_r)
    out = _trans_e(
        eids[eorder], rids[rorder],
        inv_e[:TOTAL], inv_e[TOTAL:], inv_r + jnp.int32(NENT),
        entity_emb.T, relation_emb.T,
        entity_emb[TAIL_BASE:].T, relation_emb[TAIL_BASE:].T,
    )
    return out[:BATCH], out[BATCH:]
